# baseline (speedup 1.0000x reference)
import functools
import math

import jax
import jax.numpy as jnp
from jax import lax
from jax.experimental import pallas as pl
from jax.experimental.pallas import tpu as pltpu
from jax.experimental.pallas import tpu_sc as plsc

D = 64
DP = 128
SCALE = math.sqrt(D)
NC, NS = 2, 16
NW = NC * NS

CHUNK = 512
IDX_W = 128
IDX_ROWS = CHUNK // IDX_W
STAGE = 2 * CHUNK


def _gather_body(n_stages, b_per_w, x_hbm, table_hbm, out_hbm,
                 idx_v, rows_v, sem):
    wid = lax.axis_index("s") * NC + lax.axis_index("c")
    base = wid * b_per_w

    def stage_body(g, carry):
        sbase = base + g * STAGE
        irow = pl.multiple_of(sbase // IDX_W, 8)
        pltpu.sync_copy(x_hbm.at[pl.ds(irow, STAGE // IDX_W)], idx_v)
        for half in range(2):
            cps = [
                pltpu.async_copy(
                    table_hbm.at[idx_v.at[half * IDX_ROWS + j]],
                    rows_v.at[pl.ds(j * IDX_W, IDX_W)], sem)
                for j in range(IDX_ROWS)
            ]
            for cp in cps:
                cp.wait()
            pltpu.sync_copy(
                rows_v, out_hbm.at[pl.ds(sbase + half * CHUNK, CHUNK)])
        return carry

    lax.fori_loop(0, n_stages, stage_body, 0)


def _widen_tab_body(cb, x_ref, o_ref):
    o_ref[:, :D] = jnp.transpose(x_ref[...])
    o_ref[:, D:] = jnp.zeros((cb, DP - D), jnp.float32)


def _scale_tr_body(bb, h, x_ref, o_ref):
    v3 = x_ref[...].reshape(bb, h, DP)
    for hh in range(h):
        v = v3[:, hh, :D]
        o_ref[hh] = jnp.transpose(v) * SCALE


def kernel(x, table):
    b, h = x.shape
    n = b * h
    assert n % (NW * STAGE) == 0
    b_per_w = n // NW
    n_stages = b_per_w // STAGE

    x_flat = x.reshape(n // IDX_W, IDX_W)

    v = table.shape[0]
    cb = 4096
    tabp = pl.pallas_call(
        functools.partial(_widen_tab_body, cb),
        grid=(pl.cdiv(v, cb),),
        in_specs=[pl.BlockSpec((D, cb), lambda i: (0, i))],
        out_specs=pl.BlockSpec((cb, DP), lambda i: (i, 0)),
        out_shape=jax.ShapeDtypeStruct((v, DP), jnp.float32),
    )(table.T)

    mesh = plsc.VectorSubcoreMesh(core_axis_name="c", subcore_axis_name="s")
    raw = pl.kernel(
        functools.partial(_gather_body, n_stages, b_per_w),
        mesh=mesh,
        compiler_params=pltpu.CompilerParams(use_tc_tiling_on_sc=False),
        out_type=jax.ShapeDtypeStruct((n, DP), jnp.float32),
        scratch_types=[
            pltpu.VMEM((STAGE // IDX_W, IDX_W), jnp.int32),
            pltpu.VMEM((CHUNK, DP), jnp.float32),
            pltpu.SemaphoreType.DMA,
        ],
    )(x_flat, tabp)

    bb = 256
    out_t = pl.pallas_call(
        functools.partial(_scale_tr_body, bb, h),
        grid=(b // bb,),
        in_specs=[pl.BlockSpec((bb * h, DP), lambda bi: (bi, 0))],
        out_specs=pl.BlockSpec((h, D, bb), lambda bi: (0, 0, bi)),
        out_shape=jax.ShapeDtypeStruct((h, D, b), jnp.float32),
    )(raw)
    return jnp.transpose(out_t, (2, 0, 1))


# speedup vs baseline: 1.0239x; 1.0239x over previous
import functools
import math

import jax
import jax.numpy as jnp
from jax import lax
from jax.experimental import pallas as pl
from jax.experimental.pallas import tpu as pltpu
from jax.experimental.pallas import tpu_sc as plsc

D = 64
DP = 128
SCALE = math.sqrt(D)
NC, NS = 2, 16
NW = NC * NS

CHUNK = 512
IDX_W = 128
IDX_ROWS = CHUNK // IDX_W
STAGE = 2 * CHUNK


def _gather_body(n_stages, b_per_w, x_hbm, table_hbm, out_hbm,
                 idx_v, rows_v, sem):
    wid = lax.axis_index("s") * NC + lax.axis_index("c")
    base = wid * b_per_w

    def stage_body(g, carry):
        sbase = base + g * STAGE
        irow = pl.multiple_of(sbase // IDX_W, 8)
        pltpu.sync_copy(x_hbm.at[pl.ds(irow, STAGE // IDX_W)], idx_v)
        for half in range(2):
            cps = [
                pltpu.async_copy(
                    table_hbm.at[idx_v.at[half * IDX_ROWS + j]],
                    rows_v.at[pl.ds(j * IDX_W, IDX_W)], sem)
                for j in range(IDX_ROWS)
            ]
            for cp in cps:
                cp.wait()
            pltpu.sync_copy(
                rows_v.at[:, pl.ds(0, D)],
                out_hbm.at[pl.ds(sbase + half * CHUNK, CHUNK)])
        return carry

    lax.fori_loop(0, n_stages, stage_body, 0)


def _widen_tab_body(cb, x_ref, o_ref):
    o_ref[:, :D] = jnp.transpose(x_ref[...])
    o_ref[:, D:] = jnp.zeros((cb, DP - D), jnp.float32)


def _scale_tr_body(bb, h, x_ref, o_ref):
    v3 = x_ref[...].reshape(bb, h // 2, DP)
    for j in range(h // 2):
        for k in range(2):
            v = v3[:, j, k * D:(k + 1) * D]
            o_ref[2 * j + k] = jnp.transpose(v) * SCALE


def kernel(x, table):
    b, h = x.shape
    n = b * h
    assert n % (NW * STAGE) == 0
    b_per_w = n // NW
    n_stages = b_per_w // STAGE

    x_flat = x.reshape(n // IDX_W, IDX_W)

    v = table.shape[0]
    cb = 4096
    tabp = pl.pallas_call(
        functools.partial(_widen_tab_body, cb),
        grid=(pl.cdiv(v, cb),),
        in_specs=[pl.BlockSpec((D, cb), lambda i: (0, i))],
        out_specs=pl.BlockSpec((cb, DP), lambda i: (i, 0)),
        out_shape=jax.ShapeDtypeStruct((v, DP), jnp.float32),
    )(table.T)

    mesh = plsc.VectorSubcoreMesh(core_axis_name="c", subcore_axis_name="s")
    raw = pl.kernel(
        functools.partial(_gather_body, n_stages, b_per_w),
        mesh=mesh,
        compiler_params=pltpu.CompilerParams(use_tc_tiling_on_sc=False),
        out_type=jax.ShapeDtypeStruct((n, D), jnp.float32),
        scratch_types=[
            pltpu.VMEM((STAGE // IDX_W, IDX_W), jnp.int32),
            pltpu.VMEM((CHUNK, DP), jnp.float32),
            pltpu.SemaphoreType.DMA,
        ],
    )(x_flat, tabp)

    raw2 = raw.reshape(n // 2, IDX_W)

    bb = 256
    out_t = pl.pallas_call(
        functools.partial(_scale_tr_body, bb, h),
        grid=(b // bb,),
        in_specs=[pl.BlockSpec((bb * h // 2, IDX_W), lambda bi: (bi, 0))],
        out_specs=pl.BlockSpec((h, D, bb), lambda bi: (0, 0, bi)),
        out_shape=jax.ShapeDtypeStruct((h, D, b), jnp.float32),
    )(raw2)
    return jnp.transpose(out_t, (2, 0, 1))
